# Initial kernel scaffold; baseline (speedup 1.0000x reference)
import jax
import jax.numpy as jnp
from jax.experimental import pallas as pl


def kernel(x, bn_gamma, bn_beta, bn_mean, bn_var, res_W, res_b, W_rel1, b_rel1, W_root1, W_rel2, b_rel2, W_root2, W_rel3, b_rel3, W_root3, ln1_g, ln1_b, ln2_g, ln2_b, ln3_g, ln3_b, fc_W, fc_b):
    raise NotImplementedError("write your pallas kernel here")


# speedup vs baseline: 1.0004x; 1.0004x over previous
import jax
import jax.numpy as jnp
from jax.experimental import pallas as pl

N = 10000
D = 128
H = 256
C = 527
K = 16
TW = 1.0


def _bn_body(x_ref, g_ref, b_ref, m_ref, v_ref, o_ref):
    x = x_ref[...]
    o_ref[...] = (x - m_ref[...]) / jnp.sqrt(v_ref[...] + 1e-5) * g_ref[...] + b_ref[...]


def _layer_norm(x, g, b):
    m = jnp.mean(x, axis=-1, keepdims=True)
    v = jnp.mean((x - m) ** 2, axis=-1, keepdims=True)
    return (x - m) / jnp.sqrt(v + 1e-5) * g + b


def kernel(x, bn_gamma, bn_beta, bn_mean, bn_var, res_W, res_b, W_rel1, b_rel1, W_root1, W_rel2, b_rel2, W_root2, W_rel3, b_rel3, W_root3, ln1_g, ln1_b, ln2_g, ln2_b, ln3_g, ln3_b, fc_W, fc_b):
    x = pl.pallas_call(
        _bn_body,
        out_shape=jax.ShapeDtypeStruct((N, D), jnp.float32),
    )(x, bn_gamma, bn_beta, bn_mean, bn_var)

    xn = x / (jnp.linalg.norm(x, axis=1, keepdims=True) + 1e-8)
    sim = xn @ xn.T
    _, idx = jax.lax.top_k(sim, K + 1)
    nbrs = idx[:, 1:]
    src = jnp.repeat(jnp.arange(N, dtype=jnp.int32), K)
    dst = nbrs.reshape(-1).astype(jnp.int32)
    w = sim[src, dst] + TW * (jnp.abs(dst - src) == 1).astype(jnp.float32)
    i = jnp.arange(N - 1, dtype=jnp.int32)
    present = jnp.any(nbrs[:-1].astype(jnp.int32) == (i + 1)[:, None], axis=1)
    wt = jnp.where(present, 0.0, TW).astype(jnp.float32)
    src_all = jnp.concatenate([src, i, i + 1])
    dst_all = jnp.concatenate([dst, i + 1, i])
    w_all = jnp.concatenate([w, wt, wt])

    def gconv(h, W_rel, b_rel, W_root):
        agg = jax.ops.segment_sum(h[src_all] * w_all[:, None], dst_all, num_segments=N)
        return agg @ W_rel + b_rel + h @ W_root

    r = x @ res_W + res_b
    h = gconv(x, W_rel1, b_rel1, W_root1)
    h = _layer_norm(jax.nn.relu(h) + r, ln1_g, ln1_b)
    r = h
    h = gconv(h, W_rel2, b_rel2, W_root2)
    h = _layer_norm(jax.nn.relu(h) + r, ln2_g, ln2_b)
    r = h
    h = gconv(h, W_rel3, b_rel3, W_root3)
    h = _layer_norm(jax.nn.relu(h) + r, ln3_g, ln3_b)
    return h @ fc_W + fc_b


# speedup vs baseline: 6.6659x; 6.6635x over previous
import functools

import jax
import jax.numpy as jnp
from jax.experimental import pallas as pl

N = 10000
D = 128
H = 256
C = 527
K = 16
TW = 1.0

KP1 = K + 1
RB = 400
LB = 1000

NEG = -3.0e38
BIGI = 2**30


def _simtop_body(xb_ref, xnt_ref, vals_ref, idx_ref):
    xb = xb_ref[...]
    xnt = xnt_ref[...]
    sim = jax.lax.dot_general(
        xb, xnt, (((1,), (0,)), ((), ())),
        preferred_element_type=jnp.float32)
    col = jax.lax.broadcasted_iota(jnp.int32, (RB, N), 1)

    def step(t, carry):
        sim, vals, idx = carry
        m = jnp.max(sim, axis=1)
        pos = jnp.min(jnp.where(sim == m[:, None], col, BIGI), axis=1)
        lane = jax.lax.broadcasted_iota(jnp.int32, (RB, KP1), 1)
        vals = jnp.where(lane == t, m[:, None], vals)
        idx = jnp.where(lane == t, pos[:, None], idx)
        sim = jnp.where(col == pos[:, None], NEG, sim)
        return sim, vals, idx

    vals0 = jnp.full((RB, KP1), NEG, jnp.float32)
    idx0 = jnp.zeros((RB, KP1), jnp.int32)
    _, vals, idx = jax.lax.fori_loop(0, KP1, step, (sim, vals0, idx0))
    vals_ref[...] = vals
    idx_ref[...] = idx


def _simtop(xn):
    xnt = xn.T
    return pl.pallas_call(
        _simtop_body,
        grid=(N // RB,),
        in_specs=[
            pl.BlockSpec((RB, D), lambda i: (i, 0)),
            pl.BlockSpec((D, N), lambda i: (0, 0)),
        ],
        out_specs=[
            pl.BlockSpec((RB, KP1), lambda i: (i, 0)),
            pl.BlockSpec((RB, KP1), lambda i: (i, 0)),
        ],
        out_shape=[
            jax.ShapeDtypeStruct((N, KP1), jnp.float32),
            jax.ShapeDtypeStruct((N, KP1), jnp.int32),
        ],
    )(xn, xnt)


def _segsum_jax(x, dstT, wT):
    e = dstT.shape[1]
    msgs = (wT[:, :, None] * x[:, None, :]).reshape(N * e, -1)
    return jax.ops.segment_sum(msgs, dstT.reshape(-1), num_segments=N)


def _ln(h, g, b):
    m = jnp.mean(h, axis=-1, keepdims=True)
    v = jnp.mean((h - m) ** 2, axis=-1, keepdims=True)
    return (h - m) / jnp.sqrt(v + 1e-5) * g + b


def _layer1_body(agg_ref, x_ref, wrel_ref, brel_ref, wroot_ref, resw_ref,
                 resb_ref, g_ref, b_ref, h_ref):
    x = x_ref[...]
    h = (jnp.dot(agg_ref[...], wrel_ref[...], preferred_element_type=jnp.float32)
         + brel_ref[...]
         + jnp.dot(x, wroot_ref[...], preferred_element_type=jnp.float32))
    r = jnp.dot(x, resw_ref[...], preferred_element_type=jnp.float32) + resb_ref[...]
    h_ref[...] = _ln(jax.nn.relu(h) + r, g_ref[...], b_ref[...])


def _layer_body(agg_ref, x_ref, wrel_ref, brel_ref, wroot_ref, g_ref, b_ref,
                h_ref):
    x = x_ref[...]
    h = (jnp.dot(agg_ref[...], wrel_ref[...], preferred_element_type=jnp.float32)
         + brel_ref[...]
         + jnp.dot(x, wroot_ref[...], preferred_element_type=jnp.float32))
    h_ref[...] = _ln(jax.nn.relu(h) + x, g_ref[...], b_ref[...])


def _layer1(agg, x, W_rel, b_rel, W_root, res_W, res_b, g, b):
    fin = x.shape[1]
    return pl.pallas_call(
        _layer1_body,
        grid=(N // LB,),
        in_specs=[
            pl.BlockSpec((LB, fin), lambda i: (i, 0)),
            pl.BlockSpec((LB, fin), lambda i: (i, 0)),
            pl.BlockSpec((fin, H), lambda i: (0, 0)),
            pl.BlockSpec((H,), lambda i: (0,)),
            pl.BlockSpec((fin, H), lambda i: (0, 0)),
            pl.BlockSpec((fin, H), lambda i: (0, 0)),
            pl.BlockSpec((H,), lambda i: (0,)),
            pl.BlockSpec((H,), lambda i: (0,)),
            pl.BlockSpec((H,), lambda i: (0,)),
        ],
        out_specs=pl.BlockSpec((LB, H), lambda i: (i, 0)),
        out_shape=jax.ShapeDtypeStruct((N, H), jnp.float32),
    )(agg, x, W_rel, b_rel, W_root, res_W, res_b, g, b)


def _layer(agg, x, W_rel, b_rel, W_root, g, b):
    return pl.pallas_call(
        _layer_body,
        grid=(N // LB,),
        in_specs=[
            pl.BlockSpec((LB, H), lambda i: (i, 0)),
            pl.BlockSpec((LB, H), lambda i: (i, 0)),
            pl.BlockSpec((H, H), lambda i: (0, 0)),
            pl.BlockSpec((H,), lambda i: (0,)),
            pl.BlockSpec((H, H), lambda i: (0, 0)),
            pl.BlockSpec((H,), lambda i: (0,)),
            pl.BlockSpec((H,), lambda i: (0,)),
        ],
        out_specs=pl.BlockSpec((LB, H), lambda i: (i, 0)),
        out_shape=jax.ShapeDtypeStruct((N, H), jnp.float32),
    )(agg, x, W_rel, b_rel, W_root, g, b)


def _fc_body(h_ref, w_ref, b_ref, o_ref):
    o_ref[...] = (jnp.dot(h_ref[...], w_ref[...],
                          preferred_element_type=jnp.float32) + b_ref[...])


def _fc(h, fc_W, fc_b):
    return pl.pallas_call(
        _fc_body,
        grid=(N // LB,),
        in_specs=[
            pl.BlockSpec((LB, H), lambda i: (i, 0)),
            pl.BlockSpec((H, C), lambda i: (0, 0)),
            pl.BlockSpec((C,), lambda i: (0,)),
        ],
        out_specs=pl.BlockSpec((LB, C), lambda i: (i, 0)),
        out_shape=jax.ShapeDtypeStruct((N, C), jnp.float32),
    )(h, fc_W, fc_b)


def kernel(x, bn_gamma, bn_beta, bn_mean, bn_var, res_W, res_b, W_rel1, b_rel1,
           W_root1, W_rel2, b_rel2, W_root2, W_rel3, b_rel3, W_root3,
           ln1_g, ln1_b, ln2_g, ln2_b, ln3_g, ln3_b, fc_W, fc_b):
    x = (x - bn_mean) / jnp.sqrt(bn_var + 1e-5) * bn_gamma + bn_beta
    xs = jax.lax.stop_gradient(x)
    xn = xs / (jnp.linalg.norm(xs, axis=1, keepdims=True) + 1e-8)

    vals, idx = _simtop(xn)
    nbrs = idx[:, 1:]
    v = vals[:, 1:]
    rng = jnp.arange(N, dtype=jnp.int32)
    w_knn = v + TW * (jnp.abs(nbrs - rng[:, None]) == 1).astype(jnp.float32)

    present = jnp.any(nbrs[:-1] == (rng[:-1] + 1)[:, None], axis=1)
    wt = jnp.where(present, 0.0, TW).astype(jnp.float32)
    w_fwd = jnp.concatenate([wt, jnp.zeros((1,), jnp.float32)])
    w_bwd = jnp.concatenate([jnp.zeros((1,), jnp.float32), wt])
    d_fwd = jnp.minimum(rng + 1, N - 1)
    d_bwd = jnp.maximum(rng - 1, 0)
    dstT = jnp.concatenate([nbrs, d_fwd[:, None], d_bwd[:, None]], axis=1)
    wT = jnp.concatenate([w_knn, w_fwd[:, None], w_bwd[:, None]], axis=1)

    agg1 = _segsum_jax(x, dstT, wT)
    h = _layer1(agg1, x, W_rel1, b_rel1, W_root1, res_W, res_b, ln1_g, ln1_b)
    agg2 = _segsum_jax(h, dstT, wT)
    h = _layer(agg2, h, W_rel2, b_rel2, W_root2, ln2_g, ln2_b)
    agg3 = _segsum_jax(h, dstT, wT)
    h = _layer(agg3, h, W_rel3, b_rel3, W_root3, ln3_g, ln3_b)
    return _fc(h, fc_W, fc_b)


# speedup vs baseline: 6.9329x; 1.0401x over previous
import functools

import jax
import jax.numpy as jnp
from jax.experimental import pallas as pl

N = 10000
D = 128
H = 256
C = 527
K = 16
TW = 1.0

KP1 = K + 1
RB = 400
LB = 1000

NEG = -3.0e38
BIGI = 2**30


def _simtop_body(xb_ref, xnt_ref, vals_ref, idx_ref):
    xb = xb_ref[...]
    xnt = xnt_ref[...]
    sim = jax.lax.dot_general(
        xb, xnt, (((1,), (0,)), ((), ())),
        preferred_element_type=jnp.float32)
    col = jax.lax.broadcasted_iota(jnp.int32, (RB, N), 1)

    def step(t, carry):
        vprev, pprev, vals, idx = carry
        elig = (sim < vprev[:, None]) | (
            (sim == vprev[:, None]) & (col > pprev[:, None]))
        m = jnp.max(jnp.where(elig, sim, NEG), axis=1)
        pos = jnp.min(jnp.where(elig & (sim == m[:, None]), col, BIGI), axis=1)
        lane = jax.lax.broadcasted_iota(jnp.int32, (RB, KP1), 1)
        vals = jnp.where(lane == t, m[:, None], vals)
        idx = jnp.where(lane == t, pos[:, None], idx)
        return m, pos, vals, idx

    vals0 = jnp.full((RB, KP1), NEG, jnp.float32)
    idx0 = jnp.zeros((RB, KP1), jnp.int32)
    vp0 = jnp.full((RB,), jnp.inf, jnp.float32)
    pp0 = jnp.full((RB,), -1, jnp.int32)
    _, _, vals, idx = jax.lax.fori_loop(0, KP1, step, (vp0, pp0, vals0, idx0))
    vals_ref[...] = vals
    idx_ref[...] = idx


def _simtop(xn):
    xnt = xn.T
    return pl.pallas_call(
        _simtop_body,
        grid=(N // RB,),
        in_specs=[
            pl.BlockSpec((RB, D), lambda i: (i, 0)),
            pl.BlockSpec((D, N), lambda i: (0, 0)),
        ],
        out_specs=[
            pl.BlockSpec((RB, KP1), lambda i: (i, 0)),
            pl.BlockSpec((RB, KP1), lambda i: (i, 0)),
        ],
        out_shape=[
            jax.ShapeDtypeStruct((N, KP1), jnp.float32),
            jax.ShapeDtypeStruct((N, KP1), jnp.int32),
        ],
    )(xn, xnt)


def _segsum_jax(x, dstT, wT):
    e = dstT.shape[1]
    msgs = (wT[:, :, None] * x[:, None, :]).reshape(N * e, -1)
    return jax.ops.segment_sum(msgs, dstT.reshape(-1), num_segments=N)


def _ln(h, g, b):
    m = jnp.mean(h, axis=-1, keepdims=True)
    v = jnp.mean((h - m) ** 2, axis=-1, keepdims=True)
    return (h - m) / jnp.sqrt(v + 1e-5) * g + b


def _layer1_body(agg_ref, x_ref, wrel_ref, brel_ref, wroot_ref, resw_ref,
                 resb_ref, g_ref, b_ref, h_ref):
    x = x_ref[...]
    h = (jnp.dot(agg_ref[...], wrel_ref[...], preferred_element_type=jnp.float32)
         + brel_ref[...]
         + jnp.dot(x, wroot_ref[...], preferred_element_type=jnp.float32))
    r = jnp.dot(x, resw_ref[...], preferred_element_type=jnp.float32) + resb_ref[...]
    h_ref[...] = _ln(jax.nn.relu(h) + r, g_ref[...], b_ref[...])


def _layer_body(agg_ref, x_ref, wrel_ref, brel_ref, wroot_ref, g_ref, b_ref,
                h_ref):
    x = x_ref[...]
    h = (jnp.dot(agg_ref[...], wrel_ref[...], preferred_element_type=jnp.float32)
         + brel_ref[...]
         + jnp.dot(x, wroot_ref[...], preferred_element_type=jnp.float32))
    h_ref[...] = _ln(jax.nn.relu(h) + x, g_ref[...], b_ref[...])


def _layer1(agg, x, W_rel, b_rel, W_root, res_W, res_b, g, b):
    fin = x.shape[1]
    return pl.pallas_call(
        _layer1_body,
        grid=(N // LB,),
        in_specs=[
            pl.BlockSpec((LB, fin), lambda i: (i, 0)),
            pl.BlockSpec((LB, fin), lambda i: (i, 0)),
            pl.BlockSpec((fin, H), lambda i: (0, 0)),
            pl.BlockSpec((H,), lambda i: (0,)),
            pl.BlockSpec((fin, H), lambda i: (0, 0)),
            pl.BlockSpec((fin, H), lambda i: (0, 0)),
            pl.BlockSpec((H,), lambda i: (0,)),
            pl.BlockSpec((H,), lambda i: (0,)),
            pl.BlockSpec((H,), lambda i: (0,)),
        ],
        out_specs=pl.BlockSpec((LB, H), lambda i: (i, 0)),
        out_shape=jax.ShapeDtypeStruct((N, H), jnp.float32),
    )(agg, x, W_rel, b_rel, W_root, res_W, res_b, g, b)


def _layer(agg, x, W_rel, b_rel, W_root, g, b):
    return pl.pallas_call(
        _layer_body,
        grid=(N // LB,),
        in_specs=[
            pl.BlockSpec((LB, H), lambda i: (i, 0)),
            pl.BlockSpec((LB, H), lambda i: (i, 0)),
            pl.BlockSpec((H, H), lambda i: (0, 0)),
            pl.BlockSpec((H,), lambda i: (0,)),
            pl.BlockSpec((H, H), lambda i: (0, 0)),
            pl.BlockSpec((H,), lambda i: (0,)),
            pl.BlockSpec((H,), lambda i: (0,)),
        ],
        out_specs=pl.BlockSpec((LB, H), lambda i: (i, 0)),
        out_shape=jax.ShapeDtypeStruct((N, H), jnp.float32),
    )(agg, x, W_rel, b_rel, W_root, g, b)


def _fc_body(h_ref, w_ref, b_ref, o_ref):
    o_ref[...] = (jnp.dot(h_ref[...], w_ref[...],
                          preferred_element_type=jnp.float32) + b_ref[...])


def _fc(h, fc_W, fc_b):
    return pl.pallas_call(
        _fc_body,
        grid=(N // LB,),
        in_specs=[
            pl.BlockSpec((LB, H), lambda i: (i, 0)),
            pl.BlockSpec((H, C), lambda i: (0, 0)),
            pl.BlockSpec((C,), lambda i: (0,)),
        ],
        out_specs=pl.BlockSpec((LB, C), lambda i: (i, 0)),
        out_shape=jax.ShapeDtypeStruct((N, C), jnp.float32),
    )(h, fc_W, fc_b)


def kernel(x, bn_gamma, bn_beta, bn_mean, bn_var, res_W, res_b, W_rel1, b_rel1,
           W_root1, W_rel2, b_rel2, W_root2, W_rel3, b_rel3, W_root3,
           ln1_g, ln1_b, ln2_g, ln2_b, ln3_g, ln3_b, fc_W, fc_b):
    x = (x - bn_mean) / jnp.sqrt(bn_var + 1e-5) * bn_gamma + bn_beta
    xs = jax.lax.stop_gradient(x)
    xn = xs / (jnp.linalg.norm(xs, axis=1, keepdims=True) + 1e-8)

    vals, idx = _simtop(xn)
    nbrs = idx[:, 1:]
    v = vals[:, 1:]
    rng = jnp.arange(N, dtype=jnp.int32)
    w_knn = v + TW * (jnp.abs(nbrs - rng[:, None]) == 1).astype(jnp.float32)

    present = jnp.any(nbrs[:-1] == (rng[:-1] + 1)[:, None], axis=1)
    wt = jnp.where(present, 0.0, TW).astype(jnp.float32)
    w_fwd = jnp.concatenate([wt, jnp.zeros((1,), jnp.float32)])
    w_bwd = jnp.concatenate([jnp.zeros((1,), jnp.float32), wt])
    d_fwd = jnp.minimum(rng + 1, N - 1)
    d_bwd = jnp.maximum(rng - 1, 0)
    dstT = jnp.concatenate([nbrs, d_fwd[:, None], d_bwd[:, None]], axis=1)
    wT = jnp.concatenate([w_knn, w_fwd[:, None], w_bwd[:, None]], axis=1)

    agg1 = _segsum_jax(x, dstT, wT)
    h = _layer1(agg1, x, W_rel1, b_rel1, W_root1, res_W, res_b, ln1_g, ln1_b)
    agg2 = _segsum_jax(h, dstT, wT)
    h = _layer(agg2, h, W_rel2, b_rel2, W_root2, ln2_g, ln2_b)
    agg3 = _segsum_jax(h, dstT, wT)
    h = _layer(agg3, h, W_rel3, b_rel3, W_root3, ln3_g, ln3_b)
    return _fc(h, fc_W, fc_b)
